# Initial kernel scaffold; baseline (speedup 1.0000x reference)
import jax
import jax.numpy as jnp
from jax.experimental import pallas as pl


def kernel(x, edge_index, p_w1, p_b1, l_w1, l_b1, r_w1, p_w2, p_b2, l_w2, l_b2, r_w2, p_w3, p_b3, l_w3, l_b3, r_w3):
    raise NotImplementedError("write your pallas kernel here")


# speedup vs baseline: 45.0585x; 45.0585x over previous
import functools

import jax
import jax.numpy as jnp
from jax import lax
from jax.experimental import pallas as pl
from jax.experimental.pallas import tpu as pltpu
from jax.experimental.pallas import tpu_sc as plsc

N = 100000
E = 6400000
RW = 128
ROWS = E // RW
NC, NS = 2, 16
F32 = jnp.float32

NWK = NC * NS
OUT_R = (N // NS) // 8 * 8
OUT_TAIL = N - NS * OUT_R


def _make_sc_agg(W, CH):
    NCHT = ROWS // CH
    NCH = NCHT // NWK
    NLEFT = NCHT - NCH * NWK
    mesh = plsc.VectorSubcoreMesh(core_axis_name="c", subcore_axis_name="s")

    @functools.partial(
        pl.kernel,
        out_type=jax.ShapeDtypeStruct((NC, N, W), F32),
        mesh=mesh,
        compiler_params=pltpu.CompilerParams(use_tc_tiling_on_sc=False),
        scratch_types=[
            pltpu.VMEM_SHARED((N, W), F32),
            pltpu.VMEM((CH, RW), jnp.int32),
            pltpu.VMEM((CH, RW), jnp.int32),
            pltpu.VMEM((CH, RW, W), F32),
            pltpu.SemaphoreType.DMA,
            pltpu.SemaphoreType.DMA,
        ],
    )
    def sc_agg(xp_h, src_h, dst_h, zz_h, out_h,
               agg_sp, idx_s, idx_d, rows, gsem, ssem):
        c = lax.axis_index("c")
        s = lax.axis_index("s")

        @pl.when(s == 0)
        def _():
            pltpu.sync_copy(zz_h, agg_sp)

        plsc.subcore_barrier()

        w = c * NS + s

        def chunk(k, carry):
            r0 = k * CH
            pltpu.sync_copy(src_h.at[pl.ds(r0, CH)], idx_s)
            pltpu.sync_copy(dst_h.at[pl.ds(r0, CH)], idx_d)
            gds = [pltpu.async_copy(xp_h.at[idx_s.at[j]], rows.at[j], gsem)
                   for j in range(CH)]
            for d in gds:
                d.wait()
            sds = [pltpu.async_copy(rows.at[j], agg_sp.at[idx_d.at[j]], ssem,
                                    add=True)
                   for j in range(CH)]
            for d in sds:
                d.wait()
            return carry

        lax.fori_loop(w * NCH, (w + 1) * NCH, chunk, 0)

        @pl.when(w < NLEFT)
        def _():
            chunk(NWK * NCH + w, 0)

        plsc.subcore_barrier()
        pltpu.sync_copy(agg_sp.at[pl.ds(s * OUT_R, OUT_R)],
                        out_h.at[c, pl.ds(s * OUT_R, OUT_R)])

        @pl.when(s == NS - 1)
        def _():
            pltpu.sync_copy(agg_sp.at[pl.ds(NS * OUT_R, OUT_TAIL)],
                            out_h.at[c, pl.ds(NS * OUT_R, OUT_TAIL)])

    return sc_agg


_sc_agg_w4 = _make_sc_agg(4, 16)
_sc_agg_w16 = _make_sc_agg(16, 8)


BN = 8192
_GRID = (pl.cdiv(N, BN),)


def _row_spec(d):
    return pl.BlockSpec((BN, d), lambda i: (i, 0))


def _pair_spec(d):
    return pl.BlockSpec((NC, BN, d), lambda i: (0, i, 0))


def _full_spec(a, b):
    return pl.BlockSpec((a, b), lambda i: (0, 0))


def _tc_proj1(x, w, b):
    def body(x_r, w_r, b_r, o_r):
        o_r[...] = jax.nn.relu(
            jnp.dot(x_r[...], w_r[...], preferred_element_type=F32) + b_r[...])

    return pl.pallas_call(
        body,
        grid=_GRID,
        in_specs=[_row_spec(3), _full_spec(3, 4), _full_spec(1, 4)],
        out_specs=_row_spec(4),
        out_shape=jax.ShapeDtypeStruct((N, 4), F32),
    )(x, w, b)


def _tc_epi1(p1, x, lw, lb, rw, pw2, pb2):
    def body(p_r, x_r, lw_r, lb_r, rw_r, pw_r, pb_r, h_r, xp_r):
        agg = p_r[0] + p_r[1]
        h1 = jax.nn.relu(
            jnp.dot(agg, lw_r[...], preferred_element_type=F32) + lb_r[...]
            + jnp.dot(x_r[...], rw_r[...], preferred_element_type=F32))
        h_r[...] = h1
        xp_r[...] = jax.nn.relu(
            jnp.dot(h1, pw_r[...], preferred_element_type=F32) + pb_r[...])

    return pl.pallas_call(
        body,
        grid=_GRID,
        in_specs=[_pair_spec(4), _row_spec(3), _full_spec(4, 16),
                  _full_spec(1, 16), _full_spec(3, 16), _full_spec(16, 16),
                  _full_spec(1, 16)],
        out_specs=[_row_spec(16), _row_spec(16)],
        out_shape=[jax.ShapeDtypeStruct((N, 16), F32),
                   jax.ShapeDtypeStruct((N, 16), F32)],
    )(p1, x, lw, lb, rw, pw2, pb2)


def _tc_epi2(p2, h1, lw, lb, rw, pw3, pb3):
    def body(p_r, h_r, lw_r, lb_r, rw_r, pw_r, pb_r, h2_r, xp_r):
        agg = p_r[0] + p_r[1]
        h2 = jax.nn.relu(
            jnp.dot(agg, lw_r[...], preferred_element_type=F32) + lb_r[...]
            + jnp.dot(h_r[...], rw_r[...], preferred_element_type=F32))
        h2_r[...] = h2
        xp_r[...] = jax.nn.relu(
            jnp.dot(h2, pw_r[...], preferred_element_type=F32) + pb_r[...])

    return pl.pallas_call(
        body,
        grid=_GRID,
        in_specs=[_pair_spec(16), _row_spec(16), _full_spec(16, 16),
                  _full_spec(1, 16), _full_spec(16, 16), _full_spec(16, 16),
                  _full_spec(1, 16)],
        out_specs=[_row_spec(16), _row_spec(16)],
        out_shape=[jax.ShapeDtypeStruct((N, 16), F32),
                   jax.ShapeDtypeStruct((N, 16), F32)],
    )(p2, h1, lw, lb, rw, pw3, pb3)


def _tc_epi3(p3, h2, lw, lb, rw):
    def body(p_r, h_r, lw_r, lb_r, rw_r, o_r):
        agg = p_r[0] + p_r[1]
        o_r[...] = jax.nn.sigmoid(
            jnp.dot(agg, lw_r[...], preferred_element_type=F32) + lb_r[...]
            + jnp.dot(h_r[...], rw_r[...], preferred_element_type=F32))

    return pl.pallas_call(
        body,
        grid=_GRID,
        in_specs=[_pair_spec(16), _row_spec(16), _full_spec(16, 1),
                  _full_spec(1, 1), _full_spec(16, 1)],
        out_specs=_row_spec(1),
        out_shape=jax.ShapeDtypeStruct((N, 1), F32),
    )(p3, h2, lw, lb, rw)


def kernel(x, edge_index, p_w1, p_b1, l_w1, l_b1, r_w1,
           p_w2, p_b2, l_w2, l_b2, r_w2,
           p_w3, p_b3, l_w3, l_b3, r_w3):
    src = edge_index[0].reshape(ROWS, RW)
    dst = edge_index[1].reshape(ROWS, RW)

    w1p = jnp.concatenate([p_w1, jnp.zeros((1, 3), F32)], axis=0).T
    b1p = jnp.concatenate([p_b1, jnp.zeros((1,), F32)])[None]
    lw1 = jnp.concatenate([l_w1.T, jnp.zeros((1, 16), F32)], axis=0)

    z4 = jnp.zeros((N, 4), F32)
    z16 = jnp.zeros((N, 16), F32)

    xp1 = _tc_proj1(x, w1p, b1p)
    p1 = _sc_agg_w4(xp1, src, dst, z4)
    h1, xp2 = _tc_epi1(p1, x, lw1, l_b1[None], r_w1.T,
                       p_w2.T, p_b2[None])
    p2 = _sc_agg_w16(xp2, src, dst, z16)
    h2, xp3 = _tc_epi2(p2, h1, l_w2.T, l_b2[None], r_w2.T,
                       p_w3.T, p_b3[None])
    p3 = _sc_agg_w16(xp3, src, dst, z16)
    return _tc_epi3(p3, h2, l_w3.T, l_b3[None], r_w3.T)
